# Initial kernel scaffold; baseline (speedup 1.0000x reference)
import jax
import jax.numpy as jnp
from jax.experimental import pallas as pl


def kernel(x, edge_index, edge_attr, W1, b1, W2, b2, root, bias):
    raise NotImplementedError("write your pallas kernel here")


# speedup vs baseline: 1.5340x; 1.5340x over previous
import functools

import jax
import jax.numpy as jnp
from jax import lax
from jax.experimental import pallas as pl
from jax.experimental.pallas import tpu as pltpu
from jax.experimental.pallas import tpu_sc as plsc

N_NODES = 10000
N_EDGES = 160000
IN_DIM = 128
OUT_DIM = 16
HID = 32
TCOLS = HID * OUT_DIM + OUT_DIM
TPAD = 640

NC = 2
NS = 16
NW = NC * NS

EW = N_EDGES // NW
C1 = 40
G1 = EW // C1

ET = N_EDGES // NS
C2 = 80
G2 = ET // C2
HALF0 = 5200
A_ROWS = 768
ZR2 = A_ROWS // NS
GARB_T = 6136

MB = 25
XB = 400
EB = N_EDGES // MB
HB0 = HALF0 // XB


def _prep_body(x_ref, a2_ref, ea_ref, w1_ref, b1_ref, t_ref, h_ref):
    t_ref[...] = jnp.dot(x_ref[...], a2_ref[...], preferred_element_type=jnp.float32)
    h_ref[...] = jnp.maximum(
        jnp.dot(ea_ref[...], w1_ref[...], preferred_element_type=jnp.float32) + b1_ref[...],
        0.0)


def _prep(x, a2, edge_attr, w1, b1r):
    return pl.pallas_call(
        _prep_body,
        grid=(MB,),
        in_specs=[
            pl.BlockSpec((XB, IN_DIM), lambda i: (i, 0)),
            pl.BlockSpec((IN_DIM, TPAD), lambda i: (0, 0)),
            pl.BlockSpec((EB, 16), lambda i: (i, 0)),
            pl.BlockSpec((16, HID), lambda i: (0, 0)),
            pl.BlockSpec((1, HID), lambda i: (0, 0)),
        ],
        out_specs=[
            pl.BlockSpec((XB, TPAD), lambda i: (i, 0)),
            pl.BlockSpec((EB, HID), lambda i: (i, 0)),
        ],
        out_shape=[
            jax.ShapeDtypeStruct((N_NODES, TPAD), jnp.float32),
            jax.ShapeDtypeStruct((N_EDGES, HID), jnp.float32),
        ],
    )(x, a2, edge_attr, w1, b1r)


def _msg_body(t_hbm, h_hbm, src_hbm, msg_hbm, isrc_v, h_v, rows_v, msg_v, sem):
    cid = lax.axis_index("c")
    sid = lax.axis_index("s")
    wid = cid * NS + sid

    def chunk(g, carry):
        e0 = wid * EW + g * C1
        pltpu.sync_copy(src_hbm.at[pl.ds(e0, C1)], isrc_v)
        pltpu.sync_copy(h_hbm.at[pl.ds(e0, C1)], h_v)
        pltpu.async_copy(t_hbm.at[isrc_v], rows_v, sem).wait()

        def edge(c, icarry):
            m = rows_v[c, pl.ds(HID * OUT_DIM, OUT_DIM)]
            ha = h_v[c, pl.ds(0, 16)]
            hb = h_v[c, pl.ds(16, 16)]
            for k in range(16):
                m = m + ha[k] * rows_v[c, pl.ds(k * OUT_DIM, OUT_DIM)]
            for k in range(16):
                m = m + hb[k] * rows_v[c, pl.ds((16 + k) * OUT_DIM, OUT_DIM)]
            msg_v[c, pl.ds(0, OUT_DIM)] = m
            return icarry
        lax.fori_loop(0, C1, edge, 0)

        pltpu.sync_copy(msg_v, msg_hbm.at[pl.ds(e0, C1)])
        return carry
    lax.fori_loop(0, G1, chunk, 0)


def _msg_kernel(t, h, src):
    mesh = plsc.VectorSubcoreMesh(core_axis_name="c", subcore_axis_name="s")
    f = functools.partial(
        pl.kernel,
        out_type=jax.ShapeDtypeStruct((N_EDGES, OUT_DIM), jnp.float32),
        mesh=mesh,
        scratch_types=[
            pltpu.VMEM((C1,), jnp.int32),
            pltpu.VMEM((C1, HID), jnp.float32),
            pltpu.VMEM((C1, TPAD), jnp.float32),
            pltpu.VMEM((C1, OUT_DIM), jnp.float32),
            pltpu.SemaphoreType.DMA,
        ],
    )(_msg_body)
    return f(t, h, src)


def _agg_body(dst_hbm, msg_hbm, out_hbm,
              idst_v, idx2_v, mbuf_v, mchunk_v, zero_v, accum, sem):
    cid = lax.axis_index("c")
    sid = lax.axis_index("s")
    base = cid * HALF0
    nvalid = HALF0 - cid * (2 * HALF0 - N_NODES)

    z16 = jnp.zeros((16,), jnp.float32)

    def zinit(i, carry):
        for j in range(8):
            zero_v[i, pl.ds(j * 16, 16)] = z16
        return carry
    lax.fori_loop(0, ZR2, zinit, 0)
    pltpu.sync_copy(zero_v, accum.at[pl.ds(sid * ZR2, ZR2)])

    def minit(c, carry):
        for j in range(8):
            mbuf_v[c, pl.ds(j * 16, 16)] = z16
        return carry
    lax.fori_loop(0, C2, minit, 0)

    lane = lax.iota(jnp.int32, 16)
    cntvec = jnp.where(lane == 0, 1.0, 0.0).astype(jnp.float32)

    plsc.subcore_barrier()

    def chunk_scatter(e0, is_count):
        pltpu.sync_copy(dst_hbm.at[pl.ds(e0, C2)], idst_v)
        if not is_count:
            pltpu.sync_copy(msg_hbm.at[pl.ds(e0, C2)], mchunk_v)
        slots = []
        for j in range(C2 // 16):
            d = idst_v[pl.ds(j * 16, 16)]
            t = d - base
            ok = (t >= 0) & (t < nvalid)
            tg = jnp.where(ok, t, GARB_T)
            idx2_v[pl.ds(j * 16, 16)] = lax.shift_right_logical(tg, 3)
            slotv = lax.mul(jnp.bitwise_and(tg, 7), 16)
            for l in range(16):
                c = j * 16 + l
                sl = slotv[l]
                slots.append(sl)
                if is_count:
                    mbuf_v[c, pl.ds(sl, 16)] = cntvec
                else:
                    mbuf_v[c, pl.ds(sl, 16)] = mchunk_v[c, pl.ds(0, OUT_DIM)]
        pltpu.sync_copy(mbuf_v, accum.at[idx2_v], add=True)
        for c in range(C2):
            mbuf_v[c, pl.ds(slots[c], 16)] = z16

    def cchunk(g, carry):
        chunk_scatter(sid * ET + g * C2, True)
        return carry
    lax.fori_loop(0, G2, cchunk, 0)

    plsc.subcore_barrier()
    pltpu.sync_copy(accum.at[pl.ds(sid * ZR2, ZR2)],
                    out_hbm.at[cid, 0, pl.ds(sid * ZR2, ZR2)])
    pltpu.sync_copy(zero_v, accum.at[pl.ds(sid * ZR2, ZR2)])
    plsc.subcore_barrier()

    def schunk(g, carry):
        chunk_scatter(sid * ET + g * C2, False)
        return carry
    lax.fori_loop(0, G2, schunk, 0)

    plsc.subcore_barrier()
    pltpu.sync_copy(accum.at[pl.ds(sid * ZR2, ZR2)],
                    out_hbm.at[cid, 1, pl.ds(sid * ZR2, ZR2)])


def _agg_kernel(dst, msg):
    mesh = plsc.VectorSubcoreMesh(core_axis_name="c", subcore_axis_name="s")
    f = functools.partial(
        pl.kernel,
        out_type=jax.ShapeDtypeStruct((NC, 2, A_ROWS, 128), jnp.float32),
        mesh=mesh,
        scratch_types=[
            pltpu.VMEM((C2,), jnp.int32),
            pltpu.VMEM((C2,), jnp.int32),
            pltpu.VMEM((C2, 128), jnp.float32),
            pltpu.VMEM((C2, OUT_DIM), jnp.float32),
            pltpu.VMEM((ZR2, 128), jnp.float32),
            pltpu.VMEM_SHARED((A_ROWS, 128), jnp.float32),
            pltpu.SemaphoreType.DMA,
        ],
    )(_agg_body)
    return f(dst, msg)


def _final_body(p_ref, x_ref, root_ref, bias_ref, out_ref):
    cnt = jnp.maximum(p_ref[0, 0][:, 0:1], 1.0)
    out_ref[...] = (p_ref[0, 1] / cnt
                    + jnp.dot(x_ref[...], root_ref[...], preferred_element_type=jnp.float32)
                    + bias_ref[...])


def _final(partials, x, root, biasr):
    return pl.pallas_call(
        _final_body,
        grid=(MB,),
        in_specs=[
            pl.BlockSpec((1, 2, XB, OUT_DIM),
                         lambda i: (jnp.where(i < HB0, 0, 1), 0,
                                    jnp.where(i < HB0, i, i - HB0), 0)),
            pl.BlockSpec((XB, IN_DIM), lambda i: (i, 0)),
            pl.BlockSpec((IN_DIM, OUT_DIM), lambda i: (0, 0)),
            pl.BlockSpec((1, OUT_DIM), lambda i: (0, 0)),
        ],
        out_specs=pl.BlockSpec((XB, OUT_DIM), lambda i: (i, 0)),
        out_shape=jax.ShapeDtypeStruct((N_NODES, OUT_DIM), jnp.float32),
    )(partials.reshape(NC, 2, A_ROWS * 8, OUT_DIM), x, root, biasr)


def kernel(x, edge_index, edge_attr, W1, b1, W2, b2, root, bias):
    src = edge_index[0].astype(jnp.int32)
    dst = edge_index[1].astype(jnp.int32)
    a2 = jnp.concatenate(
        [W2.reshape(HID, IN_DIM, OUT_DIM).transpose(1, 0, 2).reshape(IN_DIM, HID * OUT_DIM),
         b2.reshape(IN_DIM, OUT_DIM),
         jnp.zeros((IN_DIM, TPAD - TCOLS), jnp.float32)], axis=1)
    t, h = _prep(x, a2, edge_attr, W1, b1.reshape(1, HID))
    msg = _msg_kernel(t, h, src)
    partials = _agg_kernel(dst, msg)
    return _final(partials, x, root, bias.reshape(1, OUT_DIM))


# speedup vs baseline: 1.8333x; 1.1951x over previous
import functools

import jax
import jax.numpy as jnp
from jax import lax
from jax.experimental import pallas as pl
from jax.experimental.pallas import tpu as pltpu
from jax.experimental.pallas import tpu_sc as plsc

N_NODES = 10000
N_EDGES = 160000
IN_DIM = 128
OUT_DIM = 16
HID = 32
TCOLS = HID * OUT_DIM + OUT_DIM
TPAD = 640

NC = 2
NS = 16
NW = NC * NS

EW = N_EDGES // NW
C1 = 40
G1 = EW // C1

ET = N_EDGES // NS
C2 = 80
G2 = ET // C2
HALF0 = 5200
A_ROWS = 1408
ZR2 = A_ROWS // NS
GARB_T = 5628

MB = 25
XB = 400
EB = N_EDGES // MB
HB0 = HALF0 // XB


def _prep_body(x_ref, a2_ref, ea_ref, w1_ref, b1_ref, t_ref, h_ref):
    t_ref[...] = jnp.dot(x_ref[...], a2_ref[...], preferred_element_type=jnp.float32)
    h_ref[...] = jnp.maximum(
        jnp.dot(ea_ref[...], w1_ref[...], preferred_element_type=jnp.float32) + b1_ref[...],
        0.0)


def _prep(x, a2, edge_attr, w1, b1r):
    return pl.pallas_call(
        _prep_body,
        grid=(MB,),
        in_specs=[
            pl.BlockSpec((XB, IN_DIM), lambda i: (i, 0)),
            pl.BlockSpec((IN_DIM, TPAD), lambda i: (0, 0)),
            pl.BlockSpec((EB, 16), lambda i: (i, 0)),
            pl.BlockSpec((16, HID), lambda i: (0, 0)),
            pl.BlockSpec((1, HID), lambda i: (0, 0)),
        ],
        out_specs=[
            pl.BlockSpec((XB, TPAD), lambda i: (i, 0)),
            pl.BlockSpec((EB, HID), lambda i: (i, 0)),
        ],
        out_shape=[
            jax.ShapeDtypeStruct((N_NODES, TPAD), jnp.float32),
            jax.ShapeDtypeStruct((N_EDGES, HID), jnp.float32),
        ],
    )(x, a2, edge_attr, w1, b1r)


def _msg_body(t_hbm, h_hbm, src_hbm, msg_hbm, isrc_v, h_v, rows_v, msg_v, sem):
    cid = lax.axis_index("c")
    sid = lax.axis_index("s")
    wid = cid * NS + sid

    def chunk(g, carry):
        e0 = wid * EW + g * C1
        pltpu.sync_copy(src_hbm.at[pl.ds(e0, C1)], isrc_v)
        pltpu.sync_copy(h_hbm.at[pl.ds(e0, C1)], h_v)
        pltpu.async_copy(t_hbm.at[isrc_v], rows_v, sem).wait()

        def edge(c, icarry):
            ha = h_v[c, pl.ds(0, 16)]
            hb = h_v[c, pl.ds(16, 16)]
            acc = [rows_v[c, pl.ds(HID * OUT_DIM, OUT_DIM)]] + [None, None, None]
            for k in range(16):
                p = ha[k] * rows_v[c, pl.ds(k * OUT_DIM, OUT_DIM)]
                j = k % 4
                acc[j] = p if acc[j] is None else acc[j] + p
            for k in range(16):
                acc[k % 4] = acc[k % 4] + hb[k] * rows_v[c, pl.ds((16 + k) * OUT_DIM, OUT_DIM)]
            msg_v[c, pl.ds(0, OUT_DIM)] = (acc[0] + acc[1]) + (acc[2] + acc[3])
            return icarry
        lax.fori_loop(0, C1, edge, 0)

        pltpu.sync_copy(msg_v, msg_hbm.at[pl.ds(e0, C1)])
        return carry
    lax.fori_loop(0, G1, chunk, 0)


def _msg_kernel(t, h, src):
    mesh = plsc.VectorSubcoreMesh(core_axis_name="c", subcore_axis_name="s")
    f = functools.partial(
        pl.kernel,
        out_type=jax.ShapeDtypeStruct((N_EDGES, OUT_DIM), jnp.float32),
        mesh=mesh,
        scratch_types=[
            pltpu.VMEM((C1,), jnp.int32),
            pltpu.VMEM((C1, HID), jnp.float32),
            pltpu.VMEM((C1, TPAD), jnp.float32),
            pltpu.VMEM((C1, OUT_DIM), jnp.float32),
            pltpu.SemaphoreType.DMA,
        ],
    )(_msg_body)
    return f(t, h, src)


def _agg_body(dst_hbm, msg_hbm, out_hbm,
              idst_v, idx2_v, mbuf_v, mchunk_v, zero_v, accum, sem):
    cid = lax.axis_index("c")
    sid = lax.axis_index("s")
    base = cid * HALF0
    nvalid = HALF0 - cid * (2 * HALF0 - N_NODES)

    z16 = jnp.zeros((16,), jnp.float32)

    def zinit(i, carry):
        for j in range(8):
            zero_v[i, pl.ds(j * 16, 16)] = z16
        return carry
    lax.fori_loop(0, ZR2, zinit, 0)
    pltpu.sync_copy(zero_v, accum.at[pl.ds(sid * ZR2, ZR2)])

    def minit(c, carry):
        for j in range(8):
            mbuf_v[c, pl.ds(j * 16, 16)] = z16
        return carry
    lax.fori_loop(0, C2, minit, 0)

    lane = lax.iota(jnp.int32, 16)
    cntvec = jnp.where(lane == 0, 1.0, 0.0).astype(jnp.float32)

    plsc.subcore_barrier()

    def chunk_scatter(g, carry):
        e0 = sid * ET + g * C2
        pltpu.sync_copy(dst_hbm.at[pl.ds(e0, C2)], idst_v)
        pltpu.sync_copy(msg_hbm.at[pl.ds(e0, C2)], mchunk_v)
        slots = []
        for j in range(C2 // 16):
            d = idst_v[pl.ds(j * 16, 16)]
            t = d - base
            ok = (t >= 0) & (t < nvalid)
            tg = jnp.where(ok, t, GARB_T)
            idx2_v[pl.ds(j * 16, 16)] = lax.shift_right_logical(tg, 2)
            slotv = lax.mul(jnp.bitwise_and(tg, 3), 32)
            for l in range(16):
                c = j * 16 + l
                sl = slotv[l]
                slots.append(sl)
                mbuf_v[c, pl.ds(sl, 16)] = mchunk_v[c, pl.ds(0, OUT_DIM)]
                mbuf_v[c, pl.ds(sl + 16, 16)] = cntvec
        pltpu.sync_copy(mbuf_v, accum.at[idx2_v], add=True)
        for c in range(C2):
            mbuf_v[c, pl.ds(slots[c], 16)] = z16
            mbuf_v[c, pl.ds(slots[c] + 16, 16)] = z16
        return carry

    lax.fori_loop(0, G2, chunk_scatter, 0)

    plsc.subcore_barrier()
    pltpu.sync_copy(accum.at[pl.ds(sid * ZR2, ZR2)],
                    out_hbm.at[cid, pl.ds(sid * ZR2, ZR2)])


def _agg_kernel(dst, msg):
    mesh = plsc.VectorSubcoreMesh(core_axis_name="c", subcore_axis_name="s")
    f = functools.partial(
        pl.kernel,
        out_type=jax.ShapeDtypeStruct((NC, A_ROWS, 128), jnp.float32),
        mesh=mesh,
        scratch_types=[
            pltpu.VMEM((C2,), jnp.int32),
            pltpu.VMEM((C2,), jnp.int32),
            pltpu.VMEM((C2, 128), jnp.float32),
            pltpu.VMEM((C2, OUT_DIM), jnp.float32),
            pltpu.VMEM((ZR2, 128), jnp.float32),
            pltpu.VMEM_SHARED((A_ROWS, 128), jnp.float32),
            pltpu.SemaphoreType.DMA,
        ],
    )(_agg_body)
    return f(dst, msg)


def _final_body(p_ref, x_ref, root_ref, bias_ref, out_ref):
    p = p_ref[0]
    cnt = jnp.maximum(p[:, 16:17], 1.0)
    out_ref[...] = (p[:, 0:OUT_DIM] / cnt
                    + jnp.dot(x_ref[...], root_ref[...], preferred_element_type=jnp.float32)
                    + bias_ref[...])


def _final(partials, x, root, biasr):
    return pl.pallas_call(
        _final_body,
        grid=(MB,),
        in_specs=[
            pl.BlockSpec((1, XB, 32),
                         lambda i: (jnp.where(i < HB0, 0, 1),
                                    jnp.where(i < HB0, i, i - HB0), 0)),
            pl.BlockSpec((XB, IN_DIM), lambda i: (i, 0)),
            pl.BlockSpec((IN_DIM, OUT_DIM), lambda i: (0, 0)),
            pl.BlockSpec((1, OUT_DIM), lambda i: (0, 0)),
        ],
        out_specs=pl.BlockSpec((XB, OUT_DIM), lambda i: (i, 0)),
        out_shape=jax.ShapeDtypeStruct((N_NODES, OUT_DIM), jnp.float32),
    )(partials.reshape(NC, A_ROWS * 4, 32), x, root, biasr)


def kernel(x, edge_index, edge_attr, W1, b1, W2, b2, root, bias):
    src = edge_index[0].astype(jnp.int32)
    dst = edge_index[1].astype(jnp.int32)
    a2 = jnp.concatenate(
        [W2.reshape(HID, IN_DIM, OUT_DIM).transpose(1, 0, 2).reshape(IN_DIM, HID * OUT_DIM),
         b2.reshape(IN_DIM, OUT_DIM),
         jnp.zeros((IN_DIM, TPAD - TCOLS), jnp.float32)], axis=1)
    t, h = _prep(x, a2, edge_attr, W1, b1.reshape(1, HID))
    msg = _msg_kernel(t, h, src)
    partials = _agg_kernel(dst, msg)
    return _final(partials, x, root, bias.reshape(1, OUT_DIM))


# speedup vs baseline: 2.3206x; 1.2658x over previous
import functools

import jax
import jax.numpy as jnp
from jax import lax
from jax.experimental import pallas as pl
from jax.experimental.pallas import tpu as pltpu
from jax.experimental.pallas import tpu_sc as plsc

N_NODES = 10000
N_EDGES = 160000
IN_DIM = 128
OUT_DIM = 16
HID = 32
TCOLS = HID * OUT_DIM + OUT_DIM
TPAD = 640

NC = 2
NS = 16
NW = NC * NS

EW = N_EDGES // NW
C1 = 40
G1 = EW // C1

ET = N_EDGES // NS
C2 = 80
G2 = ET // C2
HALF0 = 5200
A_ROWS = 1408
ZR2 = A_ROWS // NS
GARB_T = 5628

MB = 25
XB = 400
EB = N_EDGES // MB
HB0 = HALF0 // XB


def _prep_body(x_ref, a2_ref, ea_ref, w1_ref, b1_ref, t_ref, h_ref):
    t_ref[...] = jnp.dot(x_ref[...], a2_ref[...], preferred_element_type=jnp.float32)
    h_ref[...] = jnp.maximum(
        jnp.dot(ea_ref[...], w1_ref[...], preferred_element_type=jnp.float32) + b1_ref[...],
        0.0)


def _prep(x, a2, edge_attr, w1, b1r):
    return pl.pallas_call(
        _prep_body,
        grid=(MB,),
        in_specs=[
            pl.BlockSpec((XB, IN_DIM), lambda i: (i, 0)),
            pl.BlockSpec((IN_DIM, TPAD), lambda i: (0, 0)),
            pl.BlockSpec((EB, 16), lambda i: (i, 0)),
            pl.BlockSpec((16, HID), lambda i: (0, 0)),
            pl.BlockSpec((1, HID), lambda i: (0, 0)),
        ],
        out_specs=[
            pl.BlockSpec((XB, TPAD), lambda i: (i, 0)),
            pl.BlockSpec((EB, HID), lambda i: (i, 0)),
        ],
        out_shape=[
            jax.ShapeDtypeStruct((N_NODES, TPAD), jnp.float32),
            jax.ShapeDtypeStruct((N_EDGES, HID), jnp.float32),
        ],
    )(x, a2, edge_attr, w1, b1r)


def _msg_body(t_hbm, h_hbm, src_hbm, msg_hbm, isrc_v, h_v, rows_v, msg_v,
              gsem0, gsem1):
    cid = lax.axis_index("c")
    sid = lax.axis_index("s")
    wid = cid * NS + sid
    ebase = wid * EW
    gsem = [gsem0, gsem1]

    def load_issue(l0, b):
        e0 = ebase + l0
        pltpu.sync_copy(src_hbm.at[pl.ds(e0, C1)], isrc_v.at[b])
        pltpu.sync_copy(h_hbm.at[pl.ds(e0, C1)], h_v.at[b])
        pltpu.async_copy(t_hbm.at[isrc_v.at[b]], rows_v.at[b], gsem[b])

    def combine_store(l0, b):
        def edge(c, icarry):
            ha = h_v[b, c, pl.ds(0, 16)]
            hb = h_v[b, c, pl.ds(16, 16)]
            acc = [rows_v[b, c, pl.ds(HID * OUT_DIM, OUT_DIM)], None, None, None]
            for k in range(16):
                p = ha[k] * rows_v[b, c, pl.ds(k * OUT_DIM, OUT_DIM)]
                j = k % 4
                acc[j] = p if acc[j] is None else acc[j] + p
            for k in range(16):
                acc[k % 4] = acc[k % 4] + hb[k] * rows_v[b, c, pl.ds((16 + k) * OUT_DIM, OUT_DIM)]
            msg_v[b, c, pl.ds(0, OUT_DIM)] = (acc[0] + acc[1]) + (acc[2] + acc[3])
            return icarry
        lax.fori_loop(0, C1, edge, 0)
        pltpu.sync_copy(msg_v.at[b], msg_hbm.at[pl.ds(ebase + l0, C1)])

    load_issue(0, 0)
    load_issue(C1, 1)

    def outer(g2, carry):
        for b in range(2):
            l0 = (2 * g2 + b) * C1
            pltpu.make_async_copy(t_hbm.at[isrc_v.at[b]], rows_v.at[b], gsem[b]).wait()
            combine_store(l0, b)
            load_issue(jnp.minimum(l0 + 2 * C1, (G1 - 1) * C1), b)
        return carry
    lax.fori_loop(0, G1 // 2, outer, 0)

    pltpu.make_async_copy(t_hbm.at[isrc_v.at[0]], rows_v.at[0], gsem[0]).wait()
    combine_store((G1 - 1) * C1, 0)
    pltpu.make_async_copy(t_hbm.at[isrc_v.at[1]], rows_v.at[1], gsem[1]).wait()


def _msg_kernel(t, h, src):
    mesh = plsc.VectorSubcoreMesh(core_axis_name="c", subcore_axis_name="s")
    f = functools.partial(
        pl.kernel,
        out_type=jax.ShapeDtypeStruct((N_EDGES, OUT_DIM), jnp.float32),
        mesh=mesh,
        scratch_types=[
            pltpu.VMEM((2, C1), jnp.int32),
            pltpu.VMEM((2, C1, HID), jnp.float32),
            pltpu.VMEM((2, C1, TPAD), jnp.float32),
            pltpu.VMEM((2, C1, OUT_DIM), jnp.float32),
            pltpu.SemaphoreType.DMA,
            pltpu.SemaphoreType.DMA,
        ],
    )(_msg_body)
    return f(t, h, src)


def _agg_body(dst_hbm, msg_hbm, out_hbm,
              idst_v, idx2_v, mbuf_v, mchunk_v, zero_v, accum, sem):
    cid = lax.axis_index("c")
    sid = lax.axis_index("s")
    base = cid * HALF0
    nvalid = HALF0 - cid * (2 * HALF0 - N_NODES)

    z16 = jnp.zeros((16,), jnp.float32)

    def zinit(i, carry):
        for j in range(8):
            zero_v[i, pl.ds(j * 16, 16)] = z16
        return carry
    lax.fori_loop(0, ZR2, zinit, 0)
    pltpu.sync_copy(zero_v, accum.at[pl.ds(sid * ZR2, ZR2)])

    def minit(c, carry):
        for j in range(8):
            mbuf_v[c, pl.ds(j * 16, 16)] = z16
        return carry
    lax.fori_loop(0, C2, minit, 0)

    lane = lax.iota(jnp.int32, 16)
    cntvec = jnp.where(lane == 0, 1.0, 0.0).astype(jnp.float32)

    plsc.subcore_barrier()

    def chunk_scatter(g, carry):
        e0 = sid * ET + g * C2
        pltpu.sync_copy(dst_hbm.at[pl.ds(e0, C2)], idst_v)
        pltpu.sync_copy(msg_hbm.at[pl.ds(e0, C2)], mchunk_v)
        slots = []
        for j in range(C2 // 16):
            d = idst_v[pl.ds(j * 16, 16)]
            t = d - base
            ok = (t >= 0) & (t < nvalid)
            tg = jnp.where(ok, t, GARB_T)
            idx2_v[pl.ds(j * 16, 16)] = lax.shift_right_logical(tg, 2)
            slotv = lax.mul(jnp.bitwise_and(tg, 3), 32)
            for l in range(16):
                c = j * 16 + l
                sl = slotv[l]
                slots.append(sl)
                mbuf_v[c, pl.ds(sl, 16)] = mchunk_v[c, pl.ds(0, OUT_DIM)]
                mbuf_v[c, pl.ds(sl + 16, 16)] = cntvec
        pltpu.sync_copy(mbuf_v, accum.at[idx2_v], add=True)
        for c in range(C2):
            mbuf_v[c, pl.ds(slots[c], 16)] = z16
            mbuf_v[c, pl.ds(slots[c] + 16, 16)] = z16
        return carry

    lax.fori_loop(0, G2, chunk_scatter, 0)

    plsc.subcore_barrier()
    pltpu.sync_copy(accum.at[pl.ds(sid * ZR2, ZR2)],
                    out_hbm.at[cid, pl.ds(sid * ZR2, ZR2)])


def _agg_kernel(dst, msg):
    mesh = plsc.VectorSubcoreMesh(core_axis_name="c", subcore_axis_name="s")
    f = functools.partial(
        pl.kernel,
        out_type=jax.ShapeDtypeStruct((NC, A_ROWS, 128), jnp.float32),
        mesh=mesh,
        scratch_types=[
            pltpu.VMEM((C2,), jnp.int32),
            pltpu.VMEM((C2,), jnp.int32),
            pltpu.VMEM((C2, 128), jnp.float32),
            pltpu.VMEM((C2, OUT_DIM), jnp.float32),
            pltpu.VMEM((ZR2, 128), jnp.float32),
            pltpu.VMEM_SHARED((A_ROWS, 128), jnp.float32),
            pltpu.SemaphoreType.DMA,
        ],
    )(_agg_body)
    return f(dst, msg)


def _final_body(p_ref, x_ref, root_ref, bias_ref, out_ref):
    p = p_ref[0]
    cnt = jnp.maximum(p[:, 16:17], 1.0)
    out_ref[...] = (p[:, 0:OUT_DIM] / cnt
                    + jnp.dot(x_ref[...], root_ref[...], preferred_element_type=jnp.float32)
                    + bias_ref[...])


def _final(partials, x, root, biasr):
    return pl.pallas_call(
        _final_body,
        grid=(MB,),
        in_specs=[
            pl.BlockSpec((1, XB, 32),
                         lambda i: (jnp.where(i < HB0, 0, 1),
                                    jnp.where(i < HB0, i, i - HB0), 0)),
            pl.BlockSpec((XB, IN_DIM), lambda i: (i, 0)),
            pl.BlockSpec((IN_DIM, OUT_DIM), lambda i: (0, 0)),
            pl.BlockSpec((1, OUT_DIM), lambda i: (0, 0)),
        ],
        out_specs=pl.BlockSpec((XB, OUT_DIM), lambda i: (i, 0)),
        out_shape=jax.ShapeDtypeStruct((N_NODES, OUT_DIM), jnp.float32),
    )(partials.reshape(NC, A_ROWS * 4, 32), x, root, biasr)


def kernel(x, edge_index, edge_attr, W1, b1, W2, b2, root, bias):
    src = edge_index[0].astype(jnp.int32)
    dst = edge_index[1].astype(jnp.int32)
    a2 = jnp.concatenate(
        [W2.reshape(HID, IN_DIM, OUT_DIM).transpose(1, 0, 2).reshape(IN_DIM, HID * OUT_DIM),
         b2.reshape(IN_DIM, OUT_DIM),
         jnp.zeros((IN_DIM, TPAD - TCOLS), jnp.float32)], axis=1)
    t, h = _prep(x, a2, edge_attr, W1, b1.reshape(1, HID))
    msg = _msg_kernel(t, h, src)
    partials = _agg_kernel(dst, msg)
    return _final(partials, x, root, bias.reshape(1, OUT_DIM))


# speedup vs baseline: 2.8740x; 1.2385x over previous
import functools

import jax
import jax.numpy as jnp
from jax import lax
from jax.experimental import pallas as pl
from jax.experimental.pallas import tpu as pltpu
from jax.experimental.pallas import tpu_sc as plsc

N_NODES = 10000
N_EDGES = 160000
IN_DIM = 128
OUT_DIM = 16
HID = 32
TCOLS = HID * OUT_DIM + OUT_DIM
TPAD = 640

NC = 2
NS = 16
NW = NC * NS

EW = N_EDGES // NW
C1 = 40
G1 = EW // C1

ET = N_EDGES // NS
C2 = 80
G2 = ET // C2
HALF0 = 5200
A_ROWS = 1408
ZR2 = A_ROWS // NS
GARB_T = 5628

MB = 25
XB = 400
EB = N_EDGES // MB
HB0 = HALF0 // XB


def _prep_body(x_ref, a2_ref, ea_ref, w1_ref, b1_ref, t_ref, h_ref):
    t_ref[...] = jnp.dot(x_ref[...], a2_ref[...], preferred_element_type=jnp.float32)
    h_ref[...] = jnp.maximum(
        jnp.dot(ea_ref[...], w1_ref[...], preferred_element_type=jnp.float32) + b1_ref[...],
        0.0)


def _prep(x, a2, edge_attr, w1, b1r):
    return pl.pallas_call(
        _prep_body,
        grid=(MB,),
        in_specs=[
            pl.BlockSpec((XB, IN_DIM), lambda i: (i, 0)),
            pl.BlockSpec((IN_DIM, TPAD), lambda i: (0, 0)),
            pl.BlockSpec((EB, 16), lambda i: (i, 0)),
            pl.BlockSpec((16, HID), lambda i: (0, 0)),
            pl.BlockSpec((1, HID), lambda i: (0, 0)),
        ],
        out_specs=[
            pl.BlockSpec((XB, TPAD), lambda i: (i, 0)),
            pl.BlockSpec((EB, HID), lambda i: (i, 0)),
        ],
        out_shape=[
            jax.ShapeDtypeStruct((N_NODES, TPAD), jnp.float32),
            jax.ShapeDtypeStruct((N_EDGES, HID), jnp.float32),
        ],
    )(x, a2, edge_attr, w1, b1r)


def _msg_body(t_hbm, h_hbm, src_hbm, msg_hbm, isrc_v, h_v, rows_v, msg_v,
              gsem0, gsem1):
    cid = lax.axis_index("c")
    sid = lax.axis_index("s")
    wid = cid * NS + sid
    ebase = wid * EW
    gsem = [gsem0, gsem1]

    def load_issue(l0, b):
        e0 = ebase + l0
        pltpu.sync_copy(src_hbm.at[pl.ds(e0, C1)], isrc_v.at[b])
        pltpu.sync_copy(h_hbm.at[pl.ds(e0, C1)], h_v.at[b])
        pltpu.async_copy(t_hbm.at[isrc_v.at[b]], rows_v.at[b], gsem[b])

    def combine_store(l0, b):
        def edge(c, icarry):
            ha = h_v[b, c, pl.ds(0, 16)]
            hb = h_v[b, c, pl.ds(16, 16)]
            acc = [rows_v[b, c, pl.ds(HID * OUT_DIM, OUT_DIM)], None, None, None]
            for k in range(16):
                p = ha[k] * rows_v[b, c, pl.ds(k * OUT_DIM, OUT_DIM)]
                j = k % 4
                acc[j] = p if acc[j] is None else acc[j] + p
            for k in range(16):
                acc[k % 4] = acc[k % 4] + hb[k] * rows_v[b, c, pl.ds((16 + k) * OUT_DIM, OUT_DIM)]
            msg_v[b, c, pl.ds(0, OUT_DIM)] = (acc[0] + acc[1]) + (acc[2] + acc[3])
            return icarry
        lax.fori_loop(0, C1, edge, 0)
        pltpu.sync_copy(msg_v.at[b], msg_hbm.at[pl.ds(ebase + l0, C1)])

    load_issue(0, 0)
    load_issue(C1, 1)

    def outer(g2, carry):
        for b in range(2):
            l0 = (2 * g2 + b) * C1
            pltpu.make_async_copy(t_hbm.at[isrc_v.at[b]], rows_v.at[b], gsem[b]).wait()
            combine_store(l0, b)
            load_issue(jnp.minimum(l0 + 2 * C1, (G1 - 1) * C1), b)
        return carry
    lax.fori_loop(0, G1 // 2, outer, 0)

    pltpu.make_async_copy(t_hbm.at[isrc_v.at[0]], rows_v.at[0], gsem[0]).wait()
    combine_store((G1 - 1) * C1, 0)
    pltpu.make_async_copy(t_hbm.at[isrc_v.at[1]], rows_v.at[1], gsem[1]).wait()


def _msg_kernel(t, h, src):
    mesh = plsc.VectorSubcoreMesh(core_axis_name="c", subcore_axis_name="s")
    f = functools.partial(
        pl.kernel,
        out_type=jax.ShapeDtypeStruct((N_EDGES, OUT_DIM), jnp.float32),
        mesh=mesh,
        scratch_types=[
            pltpu.VMEM((2, C1), jnp.int32),
            pltpu.VMEM((2, C1, HID), jnp.float32),
            pltpu.VMEM((2, C1, TPAD), jnp.float32),
            pltpu.VMEM((2, C1, OUT_DIM), jnp.float32),
            pltpu.SemaphoreType.DMA,
            pltpu.SemaphoreType.DMA,
        ],
    )(_msg_body)
    return f(t, h, src)


def _agg_body(dst_hbm, msg_hbm, out_hbm,
              idst_v, idx2_v, mbuf_v, mchunk_v, zero_v, accum, asem0, asem1):
    cid = lax.axis_index("c")
    sid = lax.axis_index("s")
    base = cid * HALF0
    nvalid = HALF0 - cid * (2 * HALF0 - N_NODES)
    ebase = sid * ET
    asem = [asem0, asem1]

    z16 = jnp.zeros((16,), jnp.float32)

    def load_issue(l0, b):
        e0 = ebase + l0
        pltpu.async_copy(dst_hbm.at[pl.ds(e0, C2)], idst_v.at[b], asem[b])
        pltpu.async_copy(msg_hbm.at[pl.ds(e0, C2)], mchunk_v.at[b], asem[b])

    def load_wait(b):
        pltpu.make_async_copy(dst_hbm.at[pl.ds(0, C2)], idst_v.at[b], asem[b]).wait()
        pltpu.make_async_copy(msg_hbm.at[pl.ds(0, C2)], mchunk_v.at[b], asem[b]).wait()

    def zinit(i, carry):
        for j in range(8):
            zero_v[i, pl.ds(j * 16, 16)] = z16
        return carry
    lax.fori_loop(0, ZR2, zinit, 0)
    pltpu.sync_copy(zero_v, accum.at[pl.ds(sid * ZR2, ZR2)])

    def minit(c, carry):
        for j in range(8):
            mbuf_v[c, pl.ds(j * 16, 16)] = z16
        return carry
    lax.fori_loop(0, C2, minit, 0)

    lane = lax.iota(jnp.int32, 16)
    cntvec = jnp.where(lane == 0, 1.0, 0.0).astype(jnp.float32)

    load_issue(0, 0)
    load_issue(C2, 1)

    plsc.subcore_barrier()

    def chunk_scatter(l0, b):
        load_wait(b)
        slots = []
        for j in range(C2 // 16):
            d = idst_v[b, pl.ds(j * 16, 16)]
            t = d - base
            ok = (t >= 0) & (t < nvalid)
            tg = jnp.where(ok, t, GARB_T)
            idx2_v[pl.ds(j * 16, 16)] = lax.shift_right_logical(tg, 2)
            slotv = lax.mul(jnp.bitwise_and(tg, 3), 32)
            for l in range(16):
                c = j * 16 + l
                sl = slotv[l]
                slots.append(sl)
                mbuf_v[c, pl.ds(sl, 16)] = mchunk_v[b, c, pl.ds(0, OUT_DIM)]
                mbuf_v[c, pl.ds(sl + 16, 16)] = cntvec
        load_issue(jnp.minimum(l0 + 2 * C2, (G2 - 1) * C2), b)
        pltpu.sync_copy(mbuf_v, accum.at[idx2_v], add=True)
        for c in range(C2):
            mbuf_v[c, pl.ds(slots[c], 16)] = z16
            mbuf_v[c, pl.ds(slots[c] + 16, 16)] = z16

    def outer(g2, carry):
        for b in range(2):
            chunk_scatter((2 * g2 + b) * C2, b)
        return carry
    lax.fori_loop(0, G2 // 2, outer, 0)

    chunk_scatter((G2 - 1) * C2, 0)
    load_wait(0)
    load_wait(1)

    plsc.subcore_barrier()
    pltpu.sync_copy(accum.at[pl.ds(sid * ZR2, ZR2)],
                    out_hbm.at[cid, pl.ds(sid * ZR2, ZR2)])


def _agg_kernel(dst, msg):
    mesh = plsc.VectorSubcoreMesh(core_axis_name="c", subcore_axis_name="s")
    f = functools.partial(
        pl.kernel,
        out_type=jax.ShapeDtypeStruct((NC, A_ROWS, 128), jnp.float32),
        mesh=mesh,
        scratch_types=[
            pltpu.VMEM((2, C2), jnp.int32),
            pltpu.VMEM((C2,), jnp.int32),
            pltpu.VMEM((C2, 128), jnp.float32),
            pltpu.VMEM((2, C2, OUT_DIM), jnp.float32),
            pltpu.VMEM((ZR2, 128), jnp.float32),
            pltpu.VMEM_SHARED((A_ROWS, 128), jnp.float32),
            pltpu.SemaphoreType.DMA,
            pltpu.SemaphoreType.DMA,
        ],
    )(_agg_body)
    return f(dst, msg)


def _final_body(p_ref, x_ref, root_ref, bias_ref, out_ref):
    p = p_ref[0]
    cnt = jnp.maximum(p[:, 16:17], 1.0)
    out_ref[...] = (p[:, 0:OUT_DIM] / cnt
                    + jnp.dot(x_ref[...], root_ref[...], preferred_element_type=jnp.float32)
                    + bias_ref[...])


def _final(partials, x, root, biasr):
    return pl.pallas_call(
        _final_body,
        grid=(MB,),
        in_specs=[
            pl.BlockSpec((1, XB, 32),
                         lambda i: (jnp.where(i < HB0, 0, 1),
                                    jnp.where(i < HB0, i, i - HB0), 0)),
            pl.BlockSpec((XB, IN_DIM), lambda i: (i, 0)),
            pl.BlockSpec((IN_DIM, OUT_DIM), lambda i: (0, 0)),
            pl.BlockSpec((1, OUT_DIM), lambda i: (0, 0)),
        ],
        out_specs=pl.BlockSpec((XB, OUT_DIM), lambda i: (i, 0)),
        out_shape=jax.ShapeDtypeStruct((N_NODES, OUT_DIM), jnp.float32),
    )(partials.reshape(NC, A_ROWS * 4, 32), x, root, biasr)


def kernel(x, edge_index, edge_attr, W1, b1, W2, b2, root, bias):
    src = edge_index[0].astype(jnp.int32)
    dst = edge_index[1].astype(jnp.int32)
    a2 = jnp.concatenate(
        [W2.reshape(HID, IN_DIM, OUT_DIM).transpose(1, 0, 2).reshape(IN_DIM, HID * OUT_DIM),
         b2.reshape(IN_DIM, OUT_DIM),
         jnp.zeros((IN_DIM, TPAD - TCOLS), jnp.float32)], axis=1)
    t, h = _prep(x, a2, edge_attr, W1, b1.reshape(1, HID))
    msg = _msg_kernel(t, h, src)
    partials = _agg_kernel(dst, msg)
    return _final(partials, x, root, bias.reshape(1, OUT_DIM))


# speedup vs baseline: 3.8168x; 1.3280x over previous
import functools

import jax
import jax.numpy as jnp
from jax import lax
from jax.experimental import pallas as pl
from jax.experimental.pallas import tpu as pltpu
from jax.experimental.pallas import tpu_sc as plsc

N_NODES = 10000
N_EDGES = 160000
IN_DIM = 128
OUT_DIM = 16
HID = 32
GCOLS = HID * OUT_DIM + OUT_DIM

NC = 2
NS = 16
NW = NC * NS

EW = N_EDGES // NW
C1 = 40
G1 = EW // C1

CB = 50
EBC = N_EDGES // CB

ET = N_EDGES // NS
C2 = 80
G2 = ET // C2
HALF0 = 5200
A_ROWS = 1408
ZR2 = A_ROWS // NS
GARB_T = 5628

MB = 25
XB = 400
EB = N_EDGES // MB
HB0 = HALF0 // XB


def _gath_body(x_hbm, src_hbm, xg_hbm, isrc_v, rows_v, gsem0, gsem1):
    cid = lax.axis_index("c")
    sid = lax.axis_index("s")
    wid = cid * NS + sid
    ebase = wid * EW
    gsem = [gsem0, gsem1]

    def load_issue(l0, b):
        e0 = ebase + l0
        pltpu.sync_copy(src_hbm.at[pl.ds(e0, C1)], isrc_v.at[b])
        pltpu.async_copy(x_hbm.at[isrc_v.at[b]], rows_v.at[b], gsem[b])

    load_issue(0, 0)
    load_issue(C1, 1)

    def outer(g2, carry):
        for b in range(2):
            l0 = (2 * g2 + b) * C1
            pltpu.make_async_copy(x_hbm.at[isrc_v.at[b]], rows_v.at[b], gsem[b]).wait()
            pltpu.sync_copy(rows_v.at[b], xg_hbm.at[pl.ds(ebase + l0, C1)])
            load_issue(jnp.minimum(l0 + 2 * C1, (G1 - 1) * C1), b)
        return carry
    lax.fori_loop(0, G1 // 2, outer, 0)

    pltpu.make_async_copy(x_hbm.at[isrc_v.at[0]], rows_v.at[0], gsem[0]).wait()
    pltpu.sync_copy(rows_v.at[0], xg_hbm.at[pl.ds(ebase + (G1 - 1) * C1, C1)])
    pltpu.make_async_copy(x_hbm.at[isrc_v.at[1]], rows_v.at[1], gsem[1]).wait()


def _gath_kernel(x, src):
    mesh = plsc.VectorSubcoreMesh(core_axis_name="c", subcore_axis_name="s")
    f = functools.partial(
        pl.kernel,
        out_type=jax.ShapeDtypeStruct((N_EDGES, IN_DIM), jnp.float32),
        mesh=mesh,
        scratch_types=[
            pltpu.VMEM((2, C1), jnp.int32),
            pltpu.VMEM((2, C1, IN_DIM), jnp.float32),
            pltpu.SemaphoreType.DMA,
            pltpu.SemaphoreType.DMA,
        ],
    )(_gath_body)
    return f(x, src)


def _combine_body(xg_ref, ea_ref, w1_ref, b1_ref, a2_ref, r_ref, s_ref, msg_ref):
    h = jnp.maximum(
        jnp.dot(ea_ref[...], w1_ref[...], preferred_element_type=jnp.float32) + b1_ref[...],
        0.0)
    g = jnp.dot(xg_ref[...], a2_ref[...], preferred_element_type=jnp.float32)
    bmat = jnp.dot(h, r_ref[...], preferred_element_type=jnp.float32)
    p = g[:, :HID * OUT_DIM] * bmat
    msg_ref[...] = (jnp.dot(p, s_ref[...], preferred_element_type=jnp.float32)
                    + g[:, HID * OUT_DIM:GCOLS])


def _combine(xg, edge_attr, w1, b1r, a2, r, s):
    return pl.pallas_call(
        _combine_body,
        grid=(CB,),
        in_specs=[
            pl.BlockSpec((EBC, IN_DIM), lambda i: (i, 0)),
            pl.BlockSpec((EBC, 16), lambda i: (i, 0)),
            pl.BlockSpec((16, HID), lambda i: (0, 0)),
            pl.BlockSpec((1, HID), lambda i: (0, 0)),
            pl.BlockSpec((IN_DIM, GCOLS), lambda i: (0, 0)),
            pl.BlockSpec((HID, HID * OUT_DIM), lambda i: (0, 0)),
            pl.BlockSpec((HID * OUT_DIM, OUT_DIM), lambda i: (0, 0)),
        ],
        out_specs=pl.BlockSpec((EBC, OUT_DIM), lambda i: (i, 0)),
        out_shape=jax.ShapeDtypeStruct((N_EDGES, OUT_DIM), jnp.float32),
    )(xg, edge_attr, w1, b1r, a2, r, s)


def _agg_body(dst_hbm, msg_hbm, out_hbm,
              idst_v, idx2_v, mbuf_v, mchunk_v, zero_v, accum, asem0, asem1):
    cid = lax.axis_index("c")
    sid = lax.axis_index("s")
    base = cid * HALF0
    nvalid = HALF0 - cid * (2 * HALF0 - N_NODES)
    ebase = sid * ET
    asem = [asem0, asem1]

    z16 = jnp.zeros((16,), jnp.float32)

    def load_issue(l0, b):
        e0 = ebase + l0
        pltpu.async_copy(dst_hbm.at[pl.ds(e0, C2)], idst_v.at[b], asem[b])
        pltpu.async_copy(msg_hbm.at[pl.ds(e0, C2)], mchunk_v.at[b], asem[b])

    def load_wait(b):
        pltpu.make_async_copy(dst_hbm.at[pl.ds(0, C2)], idst_v.at[b], asem[b]).wait()
        pltpu.make_async_copy(msg_hbm.at[pl.ds(0, C2)], mchunk_v.at[b], asem[b]).wait()

    def zinit(i, carry):
        for j in range(8):
            zero_v[i, pl.ds(j * 16, 16)] = z16
        return carry
    lax.fori_loop(0, ZR2, zinit, 0)
    pltpu.sync_copy(zero_v, accum.at[pl.ds(sid * ZR2, ZR2)])

    def minit(c, carry):
        for j in range(8):
            mbuf_v[c, pl.ds(j * 16, 16)] = z16
        return carry
    lax.fori_loop(0, C2, minit, 0)

    lane = lax.iota(jnp.int32, 16)
    cntvec = jnp.where(lane == 0, 1.0, 0.0).astype(jnp.float32)

    load_issue(0, 0)
    load_issue(C2, 1)

    plsc.subcore_barrier()

    def chunk_scatter(l0, b):
        load_wait(b)
        slots = []
        for j in range(C2 // 16):
            d = idst_v[b, pl.ds(j * 16, 16)]
            t = d - base
            ok = (t >= 0) & (t < nvalid)
            tg = jnp.where(ok, t, GARB_T)
            idx2_v[pl.ds(j * 16, 16)] = lax.shift_right_logical(tg, 2)
            slotv = lax.mul(jnp.bitwise_and(tg, 3), 32)
            for l in range(16):
                c = j * 16 + l
                sl = slotv[l]
                slots.append(sl)
                mbuf_v[c, pl.ds(sl, 16)] = mchunk_v[b, c, pl.ds(0, OUT_DIM)]
                mbuf_v[c, pl.ds(sl + 16, 16)] = cntvec
        load_issue(jnp.minimum(l0 + 2 * C2, (G2 - 1) * C2), b)
        pltpu.sync_copy(mbuf_v, accum.at[idx2_v], add=True)
        for c in range(C2):
            mbuf_v[c, pl.ds(slots[c], 16)] = z16
            mbuf_v[c, pl.ds(slots[c] + 16, 16)] = z16

    def outer(g2, carry):
        for b in range(2):
            chunk_scatter((2 * g2 + b) * C2, b)
        return carry
    lax.fori_loop(0, G2 // 2, outer, 0)

    chunk_scatter((G2 - 1) * C2, 0)
    load_wait(0)
    load_wait(1)

    plsc.subcore_barrier()
    pltpu.sync_copy(accum.at[pl.ds(sid * ZR2, ZR2)],
                    out_hbm.at[cid, pl.ds(sid * ZR2, ZR2)])


def _agg_kernel(dst, msg):
    mesh = plsc.VectorSubcoreMesh(core_axis_name="c", subcore_axis_name="s")
    f = functools.partial(
        pl.kernel,
        out_type=jax.ShapeDtypeStruct((NC, A_ROWS, 128), jnp.float32),
        mesh=mesh,
        scratch_types=[
            pltpu.VMEM((2, C2), jnp.int32),
            pltpu.VMEM((C2,), jnp.int32),
            pltpu.VMEM((C2, 128), jnp.float32),
            pltpu.VMEM((2, C2, OUT_DIM), jnp.float32),
            pltpu.VMEM((ZR2, 128), jnp.float32),
            pltpu.VMEM_SHARED((A_ROWS, 128), jnp.float32),
            pltpu.SemaphoreType.DMA,
            pltpu.SemaphoreType.DMA,
        ],
    )(_agg_body)
    return f(dst, msg)


def _final_body(p_ref, x_ref, root_ref, bias_ref, out_ref):
    p = p_ref[0]
    cnt = jnp.maximum(p[:, 16:17], 1.0)
    out_ref[...] = (p[:, 0:OUT_DIM] / cnt
                    + jnp.dot(x_ref[...], root_ref[...], preferred_element_type=jnp.float32)
                    + bias_ref[...])


def _final(partials, x, root, biasr):
    return pl.pallas_call(
        _final_body,
        grid=(MB,),
        in_specs=[
            pl.BlockSpec((1, XB, 32),
                         lambda i: (jnp.where(i < HB0, 0, 1),
                                    jnp.where(i < HB0, i, i - HB0), 0)),
            pl.BlockSpec((XB, IN_DIM), lambda i: (i, 0)),
            pl.BlockSpec((IN_DIM, OUT_DIM), lambda i: (0, 0)),
            pl.BlockSpec((1, OUT_DIM), lambda i: (0, 0)),
        ],
        out_specs=pl.BlockSpec((XB, OUT_DIM), lambda i: (i, 0)),
        out_shape=jax.ShapeDtypeStruct((N_NODES, OUT_DIM), jnp.float32),
    )(partials.reshape(NC, A_ROWS * 4, 32), x, root, biasr)


def kernel(x, edge_index, edge_attr, W1, b1, W2, b2, root, bias):
    src = edge_index[0].astype(jnp.int32)
    dst = edge_index[1].astype(jnp.int32)
    a2 = jnp.concatenate(
        [W2.reshape(HID, IN_DIM, OUT_DIM).transpose(1, 0, 2).reshape(IN_DIM, HID * OUT_DIM),
         b2.reshape(IN_DIM, OUT_DIM)], axis=1)
    kk = jnp.arange(HID)[:, None]
    jj = jnp.arange(HID * OUT_DIM)[None, :]
    r = (jj // OUT_DIM == kk).astype(jnp.float32)
    s = (jj.T % OUT_DIM == jnp.arange(OUT_DIM)[None, :]).astype(jnp.float32)
    xg = _gath_kernel(x, src)
    msg = _combine(xg, edge_attr, W1, b1.reshape(1, HID), a2, r, s)
    partials = _agg_kernel(dst, msg)
    return _final(partials, x, root, bias.reshape(1, OUT_DIM))


# speedup vs baseline: 3.8298x; 1.0034x over previous
import functools

import jax
import jax.numpy as jnp
from jax import lax
from jax.experimental import pallas as pl
from jax.experimental.pallas import tpu as pltpu
from jax.experimental.pallas import tpu_sc as plsc

N_NODES = 10000
N_EDGES = 160000
IN_DIM = 128
OUT_DIM = 16
HID = 32
GCOLS = HID * OUT_DIM + OUT_DIM

NC = 2
NS = 16
NW = NC * NS

EW = N_EDGES // NW
C1 = 40
G1 = EW // C1

CB = 50
EBC = N_EDGES // CB

ET = N_EDGES // NS
C2 = 80
G2 = ET // C2
HALF0 = 5200
A_ROWS = 1408
ZR2 = A_ROWS // NS
GARB_T = 5628

MB = 25
XB = 400
EB = N_EDGES // MB
HB0 = HALF0 // XB


def _gath_body(x_hbm, src_hbm, xg_hbm, isrc_v, rows_v, gsem0, gsem1):
    cid = lax.axis_index("c")
    sid = lax.axis_index("s")
    wid = cid * NS + sid
    ebase = wid * EW
    gsem = [gsem0, gsem1]

    def load_issue(l0, b):
        e0 = ebase + l0
        pltpu.sync_copy(src_hbm.at[pl.ds(e0, C1)], isrc_v.at[b])
        pltpu.async_copy(x_hbm.at[isrc_v.at[b]], rows_v.at[b], gsem[b])

    load_issue(0, 0)
    load_issue(C1, 1)

    def outer(g2, carry):
        for b in range(2):
            l0 = (2 * g2 + b) * C1
            pltpu.make_async_copy(x_hbm.at[isrc_v.at[b]], rows_v.at[b], gsem[b]).wait()
            pltpu.sync_copy(rows_v.at[b], xg_hbm.at[pl.ds(ebase + l0, C1)])
            load_issue(jnp.minimum(l0 + 2 * C1, (G1 - 1) * C1), b)
        return carry
    lax.fori_loop(0, G1 // 2, outer, 0)

    pltpu.make_async_copy(x_hbm.at[isrc_v.at[0]], rows_v.at[0], gsem[0]).wait()
    pltpu.sync_copy(rows_v.at[0], xg_hbm.at[pl.ds(ebase + (G1 - 1) * C1, C1)])
    pltpu.make_async_copy(x_hbm.at[isrc_v.at[1]], rows_v.at[1], gsem[1]).wait()


def _gath_kernel(x, src):
    mesh = plsc.VectorSubcoreMesh(core_axis_name="c", subcore_axis_name="s")
    f = functools.partial(
        pl.kernel,
        out_type=jax.ShapeDtypeStruct((N_EDGES, IN_DIM), jnp.float32),
        mesh=mesh,
        scratch_types=[
            pltpu.VMEM((2, C1), jnp.int32),
            pltpu.VMEM((2, C1, IN_DIM), jnp.float32),
            pltpu.SemaphoreType.DMA,
            pltpu.SemaphoreType.DMA,
        ],
    )(_gath_body)
    return f(x, src)


def _combine_body(xg_ref, ea_ref, w1_ref, b1_ref, a2_ref, r_ref, s_ref, msg_ref):
    h = jnp.maximum(
        jnp.dot(ea_ref[...], w1_ref[...], preferred_element_type=jnp.float32) + b1_ref[...],
        0.0)
    g = jnp.dot(xg_ref[...].astype(jnp.bfloat16), a2_ref[...].astype(jnp.bfloat16),
                preferred_element_type=jnp.float32)
    bmat = jnp.dot(h, r_ref[...], preferred_element_type=jnp.float32)
    p = g[:, :HID * OUT_DIM] * bmat
    msg_ref[...] = (jnp.dot(p, s_ref[...], preferred_element_type=jnp.float32)
                    + g[:, HID * OUT_DIM:GCOLS])


def _combine(xg, edge_attr, w1, b1r, a2, r, s):
    return pl.pallas_call(
        _combine_body,
        grid=(CB,),
        in_specs=[
            pl.BlockSpec((EBC, IN_DIM), lambda i: (i, 0)),
            pl.BlockSpec((EBC, 16), lambda i: (i, 0)),
            pl.BlockSpec((16, HID), lambda i: (0, 0)),
            pl.BlockSpec((1, HID), lambda i: (0, 0)),
            pl.BlockSpec((IN_DIM, GCOLS), lambda i: (0, 0)),
            pl.BlockSpec((HID, HID * OUT_DIM), lambda i: (0, 0)),
            pl.BlockSpec((HID * OUT_DIM, OUT_DIM), lambda i: (0, 0)),
        ],
        out_specs=pl.BlockSpec((EBC, OUT_DIM), lambda i: (i, 0)),
        out_shape=jax.ShapeDtypeStruct((N_EDGES, OUT_DIM), jnp.float32),
    )(xg, edge_attr, w1, b1r, a2, r, s)


def _agg_body(dst_hbm, msg_hbm, out_hbm,
              idst_v, idx2_v, mbuf_v, mchunk_v, zero_v, accum, asem0, asem1):
    cid = lax.axis_index("c")
    sid = lax.axis_index("s")
    base = cid * HALF0
    nvalid = HALF0 - cid * (2 * HALF0 - N_NODES)
    ebase = sid * ET
    asem = [asem0, asem1]

    z16 = jnp.zeros((16,), jnp.float32)

    def load_issue(l0, b):
        e0 = ebase + l0
        pltpu.async_copy(dst_hbm.at[pl.ds(e0, C2)], idst_v.at[b], asem[b])
        pltpu.async_copy(msg_hbm.at[pl.ds(e0, C2)], mchunk_v.at[b], asem[b])

    def load_wait(b):
        pltpu.make_async_copy(dst_hbm.at[pl.ds(0, C2)], idst_v.at[b], asem[b]).wait()
        pltpu.make_async_copy(msg_hbm.at[pl.ds(0, C2)], mchunk_v.at[b], asem[b]).wait()

    def zinit(i, carry):
        for j in range(8):
            zero_v[i, pl.ds(j * 16, 16)] = z16
        return carry
    lax.fori_loop(0, ZR2, zinit, 0)
    pltpu.sync_copy(zero_v, accum.at[pl.ds(sid * ZR2, ZR2)])

    def minit(c, carry):
        for j in range(8):
            mbuf_v[c, pl.ds(j * 16, 16)] = z16
        return carry
    lax.fori_loop(0, C2, minit, 0)

    lane = lax.iota(jnp.int32, 16)
    cntvec = jnp.where(lane == 0, 1.0, 0.0).astype(jnp.float32)

    load_issue(0, 0)
    load_issue(C2, 1)

    plsc.subcore_barrier()

    def chunk_scatter(l0, b):
        load_wait(b)
        slots = []
        for j in range(C2 // 16):
            d = idst_v[b, pl.ds(j * 16, 16)]
            t = d - base
            ok = (t >= 0) & (t < nvalid)
            tg = jnp.where(ok, t, GARB_T)
            idx2_v[pl.ds(j * 16, 16)] = lax.shift_right_logical(tg, 2)
            slotv = lax.mul(jnp.bitwise_and(tg, 3), 32)
            for l in range(16):
                c = j * 16 + l
                sl = slotv[l]
                slots.append(sl)
                mbuf_v[c, pl.ds(sl, 16)] = mchunk_v[b, c, pl.ds(0, OUT_DIM)]
                mbuf_v[c, pl.ds(sl + 16, 16)] = cntvec
        load_issue(jnp.minimum(l0 + 2 * C2, (G2 - 1) * C2), b)
        pltpu.sync_copy(mbuf_v, accum.at[idx2_v], add=True)
        for c in range(C2):
            mbuf_v[c, pl.ds(slots[c], 16)] = z16
            mbuf_v[c, pl.ds(slots[c] + 16, 16)] = z16

    def outer(g2, carry):
        for b in range(2):
            chunk_scatter((2 * g2 + b) * C2, b)
        return carry
    lax.fori_loop(0, G2 // 2, outer, 0)

    chunk_scatter((G2 - 1) * C2, 0)
    load_wait(0)
    load_wait(1)

    plsc.subcore_barrier()
    pltpu.sync_copy(accum.at[pl.ds(sid * ZR2, ZR2)],
                    out_hbm.at[cid, pl.ds(sid * ZR2, ZR2)])


def _agg_kernel(dst, msg):
    mesh = plsc.VectorSubcoreMesh(core_axis_name="c", subcore_axis_name="s")
    f = functools.partial(
        pl.kernel,
        out_type=jax.ShapeDtypeStruct((NC, A_ROWS, 128), jnp.float32),
        mesh=mesh,
        scratch_types=[
            pltpu.VMEM((2, C2), jnp.int32),
            pltpu.VMEM((C2,), jnp.int32),
            pltpu.VMEM((C2, 128), jnp.float32),
            pltpu.VMEM((2, C2, OUT_DIM), jnp.float32),
            pltpu.VMEM((ZR2, 128), jnp.float32),
            pltpu.VMEM_SHARED((A_ROWS, 128), jnp.float32),
            pltpu.SemaphoreType.DMA,
            pltpu.SemaphoreType.DMA,
        ],
    )(_agg_body)
    return f(dst, msg)


def _final_body(p_ref, x_ref, root_ref, bias_ref, out_ref):
    p = p_ref[0]
    cnt = jnp.maximum(p[:, 16:17], 1.0)
    out_ref[...] = (p[:, 0:OUT_DIM] / cnt
                    + jnp.dot(x_ref[...], root_ref[...], preferred_element_type=jnp.float32)
                    + bias_ref[...])


def _final(partials, x, root, biasr):
    return pl.pallas_call(
        _final_body,
        grid=(MB,),
        in_specs=[
            pl.BlockSpec((1, XB, 32),
                         lambda i: (jnp.where(i < HB0, 0, 1),
                                    jnp.where(i < HB0, i, i - HB0), 0)),
            pl.BlockSpec((XB, IN_DIM), lambda i: (i, 0)),
            pl.BlockSpec((IN_DIM, OUT_DIM), lambda i: (0, 0)),
            pl.BlockSpec((1, OUT_DIM), lambda i: (0, 0)),
        ],
        out_specs=pl.BlockSpec((XB, OUT_DIM), lambda i: (i, 0)),
        out_shape=jax.ShapeDtypeStruct((N_NODES, OUT_DIM), jnp.float32),
    )(partials.reshape(NC, A_ROWS * 4, 32), x, root, biasr)


def kernel(x, edge_index, edge_attr, W1, b1, W2, b2, root, bias):
    src = edge_index[0].astype(jnp.int32)
    dst = edge_index[1].astype(jnp.int32)
    a2 = jnp.concatenate(
        [W2.reshape(HID, IN_DIM, OUT_DIM).transpose(1, 0, 2).reshape(IN_DIM, HID * OUT_DIM),
         b2.reshape(IN_DIM, OUT_DIM)], axis=1)
    kk = jnp.arange(HID)[:, None]
    jj = jnp.arange(HID * OUT_DIM)[None, :]
    r = (jj // OUT_DIM == kk).astype(jnp.float32)
    s = (jj.T % OUT_DIM == jnp.arange(OUT_DIM)[None, :]).astype(jnp.float32)
    xg = _gath_kernel(x, src)
    msg = _combine(xg, edge_attr, W1, b1.reshape(1, HID), a2, r, s)
    partials = _agg_kernel(dst, msg)
    return _final(partials, x, root, bias.reshape(1, OUT_DIM))
